# baseline (speedup 1.0000x reference)
import functools

import jax
import jax.numpy as jnp
from jax import lax
from jax.experimental import pallas as pl
from jax.experimental.pallas import tpu as pltpu
from jax.experimental.pallas import tpu_sc as plsc


def _make_sc_broadcast(b, s, h, dtype):
    info = plsc.get_sparse_core_info()
    nc, ns = info.num_cores, info.num_subcores
    nw = nc * ns
    rows_per_w = s // nw
    chunk = 64
    n_chunks = rows_per_w // chunk
    mesh = plsc.VectorSubcoreMesh(core_axis_name="c", subcore_axis_name="s")

    @functools.partial(
        pl.kernel,
        mesh=mesh,
        out_type=jax.ShapeDtypeStruct((b, s, h), dtype),
        scratch_types=[pltpu.VMEM((chunk, h), dtype)],
    )
    def sc_broadcast(table_hbm, out_hbm, buf):
        wid = lax.axis_index("s") * nc + lax.axis_index("c")
        base = wid * rows_per_w
        for c in range(n_chunks):
            lo = base + c * chunk
            pltpu.sync_copy(table_hbm.at[pl.ds(lo, chunk)], buf)
            for bi in range(b):
                pltpu.sync_copy(buf, out_hbm.at[bi, pl.ds(lo, chunk)])

    return sc_broadcast


def kernel(x, pos_embedding):
    b = x.shape[0]
    s, h = pos_embedding.shape
    return _make_sc_broadcast(b, s, h, pos_embedding.dtype)(pos_embedding)
